# baseline (speedup 1.0000x reference)
import functools

import jax
import jax.numpy as jnp
from jax.experimental import pallas as pl
from jax.experimental.pallas import tpu as pltpu


def _vq_body(flat_ref, cb_ref, w_ref, q_ref, codes_ref, probs_ref, loss_ref):
    f = flat_ref[:]
    cb = cb_ref[:]
    w = w_ref[:]
    cb_size = cb.shape[0]

    m2 = jax.lax.dot_general(
        f + f, cb, (((1,), (1,)), ((), ())), preferred_element_type=jnp.float32
    )
    fn = jnp.sum(f * f, axis=1, keepdims=True)
    cn = jnp.sum(cb * cb, axis=1)
    d = (fn - m2) + cn[None, :]

    dmin = jnp.min(d, axis=1, keepdims=True)
    e = jnp.exp(dmin - d)
    ssum = jnp.sum(e, axis=1, keepdims=True)
    probs_ref[:] = e * (1.0 / ssum)

    oh = jnp.where(d == dmin, 1.0, 0.0)
    aux = jax.lax.dot_general(
        oh, w, (((1,), (0,)), ((), ())), preferred_element_type=jnp.float32
    )
    nhot = aux[:, 0:1]
    has_tie = jnp.max(nhot) > 1.5

    @pl.when(jnp.logical_not(has_tie))
    def _fast():
        codes_f = aux[:, 1:2] * 4.0 + aux[:, 2:3]
        codes_ref[:] = codes_f[:, 0].astype(jnp.int32)
        q_ref[:] = jax.lax.dot_general(
            oh, cb, (((1,), (0,)), ((), ())), preferred_element_type=jnp.float32
        )

    @pl.when(has_tie)
    def _tie():
        iota_f = jax.lax.broadcasted_iota(
            jnp.int32, d.shape, 1).astype(jnp.float32)
        cand = jnp.where(d == dmin, iota_f, float(cb_size))
        codes_f = jnp.min(cand, axis=1, keepdims=True)
        codes_ref[:] = codes_f[:, 0].astype(jnp.int32)
        oh1 = (iota_f == codes_f).astype(jnp.float32)
        q_ref[:] = jax.lax.dot_general(
            oh1, cb, (((1,), (0,)), ((), ())),
            preferred_element_type=jnp.float32,
        )

    part = jnp.sum(dmin).reshape(1, 1)
    i = pl.program_id(0)

    @pl.when(i == 0)
    def _init():
        loss_ref[:] = part

    @pl.when(i > 0)
    def _acc():
        loss_ref[:] = loss_ref[:] + part


@functools.partial(jax.jit, static_argnames=("tile",))
def _vq_pallas(flat, codebook, tile=2048):
    n, d = flat.shape
    cb_size = codebook.shape[0]
    idx = jnp.arange(cb_size, dtype=jnp.float32)
    w = jnp.stack(
        [jnp.ones(cb_size, jnp.float32), jnp.floor(idx / 4.0), idx % 4.0],
        axis=1,
    )
    w = jnp.pad(w, ((0, 0), (0, 125)))
    grid = (n // tile,)
    q, codes, probs, loss = pl.pallas_call(
        _vq_body,
        grid=grid,
        in_specs=[
            pl.BlockSpec((tile, d), lambda i: (i, 0)),
            pl.BlockSpec((cb_size, d), lambda i: (0, 0)),
            pl.BlockSpec((cb_size, 128), lambda i: (0, 0)),
        ],
        out_specs=[
            pl.BlockSpec((tile, d), lambda i: (i, 0)),
            pl.BlockSpec((tile,), lambda i: (i,)),
            pl.BlockSpec((tile, cb_size), lambda i: (i, 0)),
            pl.BlockSpec((1, 1), lambda i: (0, 0)),
        ],
        out_shape=[
            jax.ShapeDtypeStruct((n, d), jnp.float32),
            jax.ShapeDtypeStruct((n,), jnp.int32),
            jax.ShapeDtypeStruct((n, cb_size), jnp.float32),
            jax.ShapeDtypeStruct((1, 1), jnp.float32),
        ],
    )(flat, codebook, w)
    return q, codes, probs, loss


def kernel(slot_embeddings, codebook):
    batch, k, d = slot_embeddings.shape
    cb_size = codebook.shape[0]
    flat = slot_embeddings.reshape(-1, d)
    q, codes, probs, loss = _vq_pallas(flat, codebook)
    beta = 0.25
    vq_loss = ((1.0 + beta) * loss[0, 0] / (flat.shape[0] * d)).astype(jnp.float32)
    return (
        q.reshape(batch, k, d),
        codes.reshape(batch, k),
        probs.reshape(batch, k, cb_size),
        vq_loss,
    )


# speedup vs baseline: 1.1182x; 1.1182x over previous
import functools

import jax
import jax.numpy as jnp
from jax.experimental import pallas as pl
from jax.experimental.pallas import tpu as pltpu


def _vq_body(flat_ref, cb_ref, q_ref, codes_ref, probs_ref, loss_ref):
    f = flat_ref[:]
    cb = cb_ref[:]
    cb_size = cb.shape[0]

    m2 = jax.lax.dot_general(
        f + f, cb, (((1,), (1,)), ((), ())), preferred_element_type=jnp.float32
    )
    fn = jnp.sum(f * f, axis=1, keepdims=True)
    cn = jnp.sum(cb * cb, axis=1)
    d = (fn - m2) + cn[None, :]

    dmin = jnp.min(d, axis=1, keepdims=True)
    e = jnp.exp(dmin - d)
    ssum = jnp.sum(e, axis=1, keepdims=True)
    probs_ref[:] = e * (1.0 / ssum)

    iota_f = jax.lax.broadcasted_iota(jnp.int32, d.shape, 1).astype(jnp.float32)
    cand = jnp.where(d == dmin, iota_f, float(cb_size))
    codes_f = jnp.min(cand, axis=1, keepdims=True)
    codes_ref[:] = codes_f.astype(jnp.int32)

    oh = (iota_f == codes_f).astype(jnp.float32)
    q_ref[:] = jax.lax.dot_general(
        oh, cb, (((1,), (0,)), ((), ())), preferred_element_type=jnp.float32
    )

    part = jnp.sum(dmin).reshape(1, 1)
    i = pl.program_id(0)

    @pl.when(i == 0)
    def _init():
        loss_ref[:] = part

    @pl.when(i > 0)
    def _acc():
        loss_ref[:] = loss_ref[:] + part


@functools.partial(jax.jit, static_argnames=("tile",))
def _vq_pallas(flat, codebook, tile=2048):
    n, d = flat.shape
    cb_size = codebook.shape[0]
    grid = (n // tile,)
    q, codes, probs, loss = pl.pallas_call(
        _vq_body,
        grid=grid,
        in_specs=[
            pl.BlockSpec((tile, d), lambda i: (i, 0)),
            pl.BlockSpec((cb_size, d), lambda i: (0, 0)),
        ],
        out_specs=[
            pl.BlockSpec((tile, d), lambda i: (i, 0)),
            pl.BlockSpec((tile, 1), lambda i: (i, 0)),
            pl.BlockSpec((tile, cb_size), lambda i: (i, 0)),
            pl.BlockSpec((1, 1), lambda i: (0, 0)),
        ],
        out_shape=[
            jax.ShapeDtypeStruct((n, d), jnp.float32),
            jax.ShapeDtypeStruct((n, 1), jnp.int32),
            jax.ShapeDtypeStruct((n, cb_size), jnp.float32),
            jax.ShapeDtypeStruct((1, 1), jnp.float32),
        ],
    )(flat, codebook)
    return q, codes, probs, loss


def kernel(slot_embeddings, codebook):
    batch, k, d = slot_embeddings.shape
    cb_size = codebook.shape[0]
    flat = slot_embeddings.reshape(-1, d)
    q, codes, probs, loss = _vq_pallas(flat, codebook)
    beta = 0.25
    vq_loss = ((1.0 + beta) * loss[0, 0] / (flat.shape[0] * d)).astype(jnp.float32)
    return (
        q.reshape(batch, k, d),
        codes.reshape(batch, k),
        probs.reshape(batch, k, cb_size),
        vq_loss,
    )
